# baseline (speedup 1.0000x reference)
import functools

import jax
import jax.numpy as jnp
from jax import lax
from jax.experimental import pallas as pl
from jax.experimental.pallas import tpu as pltpu, tpu_sc as plsc

NUM_FIELDS = 26
VOCAB = 100000
EMB = 32
GROUP = 8
QUAD_D = GROUP * EMB
NUM_QUADS = 4
BATCH = 16384
OUT_D = NUM_FIELDS * EMB
OUT_PAD = NUM_QUADS * QUAD_D

_info = plsc.get_sparse_core_info()
_NC, _NS = _info.num_cores, _info.num_subcores
_NW = _NC * _NS
_BPW = BATCH // _NW
_CHUNK = 512
_NCHUNK = _BPW // _CHUNK

_mesh = plsc.VectorSubcoreMesh(core_axis_name="c", subcore_axis_name="s")


@functools.partial(
    pl.kernel,
    mesh=_mesh,
    out_type=jax.ShapeDtypeStruct((BATCH, OUT_PAD), jnp.float32),
    compiler_params=pltpu.CompilerParams(use_tc_tiling_on_sc=False),
    scratch_types=[
        pltpu.VMEM((NUM_FIELDS, _BPW), jnp.int32),
        pltpu.VMEM((_CHUNK, EMB), jnp.float32),
        pltpu.VMEM((_CHUNK, EMB), jnp.float32),
        pltpu.SemaphoreType.DMA,
        pltpu.SemaphoreType.DMA,
        pltpu.SemaphoreType.DMA,
    ],
)
def _gather_concat(*refs):
    idx_refs = refs[:NUM_FIELDS]
    quads = refs[NUM_FIELDS:NUM_FIELDS + NUM_QUADS]
    out_hbm = refs[NUM_FIELDS + NUM_QUADS]
    idx_v, buf0, buf1, gsem, wsem0, wsem1 = refs[NUM_FIELDS + NUM_QUADS + 1:]

    wid = lax.axis_index("s") * _NC + lax.axis_index("c")
    base = wid * _BPW

    for f in range(NUM_FIELDS):
        pltpu.sync_copy(idx_refs[f].at[pl.ds(base, _BPW)], idx_v.at[f])

    bufs = (buf0, buf1)
    wsems = (wsem0, wsem1)
    writes = [None, None]
    step = 0
    for f in range(NUM_FIELDS):
        for c in range(_NCHUNK):
            b = step & 1
            step += 1
            if writes[b] is not None:
                writes[b].wait()
            pltpu.async_copy(
                quads[f // GROUP].at[idx_v.at[f, pl.ds(c * _CHUNK, _CHUNK)]],
                bufs[b],
                gsem,
            ).wait()
            writes[b] = pltpu.async_copy(
                bufs[b],
                out_hbm.at[
                    pl.ds(base + c * _CHUNK, _CHUNK), pl.ds(f * EMB, EMB)
                ],
                wsems[b],
            )
    for b in range(2):
        if writes[b] is not None:
            writes[b].wait()


def kernel(
    field_00, field_01, field_02, field_03, field_04, field_05, field_06,
    field_07, field_08, field_09, field_10, field_11, field_12, field_13,
    field_14, field_15, field_16, field_17, field_18, field_19, field_20,
    field_21, field_22, field_23, field_24, field_25,
    W_field_00, W_field_01, W_field_02, W_field_03, W_field_04, W_field_05,
    W_field_06, W_field_07, W_field_08, W_field_09, W_field_10, W_field_11,
    W_field_12, W_field_13, W_field_14, W_field_15, W_field_16, W_field_17,
    W_field_18, W_field_19, W_field_20, W_field_21, W_field_22, W_field_23,
    W_field_24, W_field_25,
):
    fields = (
        field_00, field_01, field_02, field_03, field_04, field_05, field_06,
        field_07, field_08, field_09, field_10, field_11, field_12, field_13,
        field_14, field_15, field_16, field_17, field_18, field_19, field_20,
        field_21, field_22, field_23, field_24, field_25,
    )
    tables = [
        W_field_00, W_field_01, W_field_02, W_field_03, W_field_04, W_field_05,
        W_field_06, W_field_07, W_field_08, W_field_09, W_field_10, W_field_11,
        W_field_12, W_field_13, W_field_14, W_field_15, W_field_16, W_field_17,
        W_field_18, W_field_19, W_field_20, W_field_21, W_field_22, W_field_23,
        W_field_24, W_field_25,
    ]
    zero = jnp.zeros((VOCAB, EMB), jnp.float32)
    tables += [zero] * (NUM_QUADS * GROUP - NUM_FIELDS)
    quads = tuple(
        jnp.concatenate(
            tables[g * GROUP:(g + 1) * GROUP], axis=1
        ).reshape(GROUP * VOCAB, EMB)
        for g in range(NUM_QUADS)
    )
    fields4 = tuple(
        f_i * GROUP + (i % GROUP) for i, f_i in enumerate(fields)
    )
    return _gather_concat(*fields4, *quads)[:, :OUT_D]


# speedup vs baseline: 1.6072x; 1.6072x over previous
import functools

import jax
import jax.numpy as jnp
from jax import lax
from jax.experimental import pallas as pl
from jax.experimental.pallas import tpu as pltpu, tpu_sc as plsc

NUM_FIELDS = 26
VOCAB = 100000
EMB = 32
GROUP = 4
QUAD_D = GROUP * EMB
NUM_QUADS = 7
BATCH = 16384
OUT_D = NUM_FIELDS * EMB
OUT_PAD = NUM_QUADS * QUAD_D

_info = plsc.get_sparse_core_info()
_NC, _NS = _info.num_cores, _info.num_subcores
_NW = _NC * _NS
_BPW = BATCH // _NW
_CHUNK = 512
_NCHUNK = _BPW // _CHUNK

_mesh = plsc.VectorSubcoreMesh(core_axis_name="c", subcore_axis_name="s")


@functools.partial(
    pl.kernel,
    mesh=_mesh,
    out_type=jax.ShapeDtypeStruct((BATCH, OUT_PAD), jnp.float32),
    compiler_params=pltpu.CompilerParams(use_tc_tiling_on_sc=False),
    scratch_types=[
        pltpu.VMEM((NUM_FIELDS, _BPW), jnp.int32),
        pltpu.VMEM((_CHUNK, EMB), jnp.float32),
        pltpu.VMEM((_CHUNK, EMB), jnp.float32),
        pltpu.SemaphoreType.DMA,
        pltpu.SemaphoreType.DMA,
        pltpu.SemaphoreType.DMA,
    ],
)
def _gather_concat(*refs):
    idx_refs = refs[:NUM_FIELDS]
    quads = refs[NUM_FIELDS:NUM_FIELDS + NUM_QUADS]
    out_hbm = refs[NUM_FIELDS + NUM_QUADS]
    idx_v, buf0, buf1, gsem, wsem0, wsem1 = refs[NUM_FIELDS + NUM_QUADS + 1:]

    wid = lax.axis_index("s") * _NC + lax.axis_index("c")
    base = wid * _BPW

    for f in range(NUM_FIELDS):
        pltpu.sync_copy(idx_refs[f].at[pl.ds(base, _BPW)], idx_v.at[f])

    bufs = (buf0, buf1)
    wsems = (wsem0, wsem1)
    writes = [None, None]
    step = 0
    for f in range(NUM_FIELDS):
        for c in range(_NCHUNK):
            b = step & 1
            step += 1
            if writes[b] is not None:
                writes[b].wait()
            pltpu.async_copy(
                quads[f // GROUP].at[idx_v.at[f, pl.ds(c * _CHUNK, _CHUNK)]],
                bufs[b],
                gsem,
            ).wait()
            writes[b] = pltpu.async_copy(
                bufs[b],
                out_hbm.at[
                    pl.ds(base + c * _CHUNK, _CHUNK), pl.ds(f * EMB, EMB)
                ],
                wsems[b],
            )
    for b in range(2):
        if writes[b] is not None:
            writes[b].wait()


def kernel(
    field_00, field_01, field_02, field_03, field_04, field_05, field_06,
    field_07, field_08, field_09, field_10, field_11, field_12, field_13,
    field_14, field_15, field_16, field_17, field_18, field_19, field_20,
    field_21, field_22, field_23, field_24, field_25,
    W_field_00, W_field_01, W_field_02, W_field_03, W_field_04, W_field_05,
    W_field_06, W_field_07, W_field_08, W_field_09, W_field_10, W_field_11,
    W_field_12, W_field_13, W_field_14, W_field_15, W_field_16, W_field_17,
    W_field_18, W_field_19, W_field_20, W_field_21, W_field_22, W_field_23,
    W_field_24, W_field_25,
):
    fields = (
        field_00, field_01, field_02, field_03, field_04, field_05, field_06,
        field_07, field_08, field_09, field_10, field_11, field_12, field_13,
        field_14, field_15, field_16, field_17, field_18, field_19, field_20,
        field_21, field_22, field_23, field_24, field_25,
    )
    tables = [
        W_field_00, W_field_01, W_field_02, W_field_03, W_field_04, W_field_05,
        W_field_06, W_field_07, W_field_08, W_field_09, W_field_10, W_field_11,
        W_field_12, W_field_13, W_field_14, W_field_15, W_field_16, W_field_17,
        W_field_18, W_field_19, W_field_20, W_field_21, W_field_22, W_field_23,
        W_field_24, W_field_25,
    ]
    zero = jnp.zeros((VOCAB, EMB), jnp.float32)
    tables += [zero, zero]
    quads = tuple(
        jnp.concatenate(
            tables[g * GROUP:(g + 1) * GROUP], axis=1
        ).reshape(GROUP * VOCAB, EMB)
        for g in range(NUM_QUADS)
    )
    fields4 = tuple(
        f_i * GROUP + (i % GROUP) for i, f_i in enumerate(fields)
    )
    return _gather_concat(*fields4, *quads)[:, :OUT_D]
